# Initial kernel scaffold; baseline (speedup 1.0000x reference)
import jax
import jax.numpy as jnp
from jax.experimental import pallas as pl


def kernel(x, src0, dst0, W0, b0, src1, dst1, W1, b1, src2, dst2, W2, b2, src3, dst3, W3, b3, src4, dst4, W4, b4, W_fc1, b_fc1, W_fc2, b_fc2):
    raise NotImplementedError("write your pallas kernel here")


# speedup vs baseline: 46.7967x; 46.7967x over previous
import functools

import jax
import jax.numpy as jnp
from jax import lax
from jax.experimental import pallas as pl
from jax.experimental.pallas import tpu as pltpu
from jax.experimental.pallas import tpu_sc as plsc

_NODE = [100000, 65536, 16384, 4096, 1024, 128]
_CH = [1, 8, 16, 32, 64, 128]
_B = 8
_NSUB = 16
_NW = 2 * _NSUB
_CHUNK = 128
_TRASH = 128


def _prep_edges(src, dst, n_out):
    E = src.shape[0]
    per = E // _NW
    nk = -(-per // _CHUNK)
    pad = nk * _CHUNK - per
    s = src.reshape(_NW, per)
    d = dst.reshape(_NW, per)
    if pad:
        s = jnp.pad(s, ((0, 0), (0, pad)))
        trash = n_out + (jnp.arange(pad, dtype=jnp.int32) % _TRASH)
        d = jnp.concatenate(
            [d, jnp.broadcast_to(trash, (_NW, pad))], axis=1)
    return s.reshape(_NW, nk, _CHUNK), d.reshape(_NW, nk, _CHUNK), nk


def _sc_gather_scatter(n_in, n_out, R, nk, z, src_r, dst_r, zeros):
    rows_pad = n_out + _TRASH
    rpt_zero = rows_pad // _NSUB
    rpt_out = n_out // _NSUB
    mesh = plsc.VectorSubcoreMesh(core_axis_name="c", subcore_axis_name="s")

    @functools.partial(
        pl.kernel,
        mesh=mesh,
        compiler_params=pltpu.CompilerParams(use_tc_tiling_on_sc=False),
        out_type=jax.ShapeDtypeStruct((2, n_out, R), jnp.float32),
        scratch_types=[
            pltpu.VMEM((1, _CHUNK), jnp.int32),
            pltpu.VMEM((1, _CHUNK), jnp.int32),
            pltpu.VMEM((_CHUNK, R), jnp.float32),
            pltpu.VMEM_SHARED((rows_pad, R), jnp.float32),
        ],
    )
    def k(z_hbm, src_hbm, dst_hbm, zero_hbm, out_hbm, sidx, didx, staged, agg):
        c = lax.axis_index("c")
        s = lax.axis_index("s")
        w = c * _NSUB + s
        pltpu.sync_copy(zero_hbm.at[pl.ds(s * rpt_zero, rpt_zero)],
                        agg.at[pl.ds(s * rpt_zero, rpt_zero)])
        plsc.subcore_barrier()

        @pl.loop(0, nk)
        def _(kk):
            pltpu.sync_copy(src_hbm.at[w, kk], sidx.at[0])
            pltpu.sync_copy(dst_hbm.at[w, kk], didx.at[0])
            pltpu.sync_copy(z_hbm.at[sidx.at[0]], staged)
            pltpu.sync_copy(staged, agg.at[didx.at[0]], add=True)

        plsc.subcore_barrier()
        pltpu.sync_copy(agg.at[pl.ds(s * rpt_out, rpt_out)],
                        out_hbm.at[c, pl.ds(s * rpt_out, rpt_out)])

    return k(z, src_r, dst_r, zeros)


def _mix(p, M, bvec, bn):
    n_out, Rin = p.shape[1], p.shape[2]
    Rout = M.shape[1]

    def body(p_ref, m_ref, b_ref, o_ref):
        a = p_ref[0] + p_ref[1]
        o_ref[...] = (
            jnp.dot(a, m_ref[...], preferred_element_type=jnp.float32)
            + b_ref[...])

    return pl.pallas_call(
        body,
        grid=(n_out // bn,),
        in_specs=[
            pl.BlockSpec((2, bn, Rin), lambda i: (0, i, 0)),
            pl.BlockSpec((Rin, Rout), lambda i: (0, 0)),
            pl.BlockSpec((1, Rout), lambda i: (0, 0)),
        ],
        out_specs=pl.BlockSpec((bn, Rout), lambda i: (i, 0)),
        out_shape=jax.ShapeDtypeStruct((n_out, Rout), jnp.float32),
    )(p, M, bvec)


def _final(p4, M4, bv4, W1T, b1, W2T, b2):
    def body(p_ref, m_ref, bv_ref, w1_ref, b1_ref, w2_ref, b2_ref, o_ref):
        z5 = (jnp.dot(p_ref[0] + p_ref[1], m_ref[...],
                      preferred_element_type=jnp.float32) + bv_ref[...])
        flat = z5.T.reshape(_B, 128 * 128)
        h = (jnp.dot(flat, w1_ref[...], preferred_element_type=jnp.float32)
             + b1_ref[...])
        o_ref[...] = (
            jnp.dot(h, w2_ref[...], preferred_element_type=jnp.float32)
            + b2_ref[...])

    return pl.pallas_call(
        body,
        out_shape=jax.ShapeDtypeStruct((_B, 10), jnp.float32),
    )(p4, M4, bv4, W1T, b1, W2T, b2)


def kernel(x, src0, dst0, W0, b0, src1, dst1, W1, b1, src2, dst2, W2, b2,
           src3, dst3, W3, b3, src4, dst4, W4, b4, W_fc1, b_fc1, W_fc2, b_fc2):
    srcs = [src0, src1, src2, src3, src4]
    dsts = [dst0, dst1, dst2, dst3, dst4]
    Ws = [W0, W1, W2, W3, W4]
    bs = [b0, b1, b2, b3, b4]

    eye = jnp.eye(_B, dtype=jnp.float32)
    z = x.T
    partials = None
    for l in range(5):
        n_in, n_out = _NODE[l], _NODE[l + 1]
        R_in = _B * _CH[l]
        src_r, dst_r, nk = _prep_edges(srcs[l], dsts[l], n_out)
        zeros = jnp.zeros((n_out + _TRASH, R_in), jnp.float32)
        partials = _sc_gather_scatter(n_in, n_out, R_in, nk,
                                      z, src_r, dst_r, zeros)
        if l < 4:
            M = jnp.kron(eye, Ws[l].T)
            bvec = jnp.tile(bs[l], _B)[None, :]
            bn = min(n_out, 8192)
            z = _mix(partials, M, bvec, bn)

    M4 = jnp.kron(eye, W4.T)
    bv4 = jnp.tile(b4, _B)[None, :]
    out = _final(partials, M4, bv4,
                 W_fc1.T, b_fc1[None, :], W_fc2.T, b_fc2[None, :])
    return out


# speedup vs baseline: 70.2019x; 1.5001x over previous
import functools

import jax
import jax.numpy as jnp
from jax import lax
from jax.experimental import pallas as pl
from jax.experimental.pallas import tpu as pltpu
from jax.experimental.pallas import tpu_sc as plsc

_NODE = [100000, 65536, 16384, 4096, 1024, 128]
_CH = [1, 8, 16, 32, 64, 128]
_B = 8
_NSUB = 16
_NW = 2 * _NSUB
_CHUNK = 128
_TRASH = 128


def _prep_edges(src, dst, n_out):
    E = src.shape[0]
    per = E // _NW
    chunk = _CHUNK if per >= 2 * _CHUNK else max(8, per // 2)
    nk = -(-per // chunk)
    nk += nk % 2
    pad = nk * chunk - per
    s = src.reshape(_NW, per)
    d = dst.reshape(_NW, per)
    if pad:
        s = jnp.pad(s, ((0, 0), (0, pad)))
        trash = n_out + (jnp.arange(pad, dtype=jnp.int32) % _TRASH)
        d = jnp.concatenate(
            [d, jnp.broadcast_to(trash, (_NW, pad))], axis=1)
    return s.reshape(_NW, nk, chunk), d.reshape(_NW, nk, chunk), nk, chunk


def _sc_gather_scatter(n_in, n_out, R, nk, chunk, z, src_r, dst_r, zeros):
    rows_pad = n_out + _TRASH
    rpt_zero = rows_pad // _NSUB
    rpt_out = n_out // _NSUB
    mesh = plsc.VectorSubcoreMesh(core_axis_name="c", subcore_axis_name="s")

    @functools.partial(
        pl.kernel,
        mesh=mesh,
        compiler_params=pltpu.CompilerParams(use_tc_tiling_on_sc=False),
        out_type=jax.ShapeDtypeStruct((2, n_out, R), jnp.float32),
        scratch_types=[
            pltpu.VMEM((nk, chunk), jnp.int32),
            pltpu.VMEM((nk, chunk), jnp.int32),
            pltpu.VMEM((chunk, R), jnp.float32),
            pltpu.VMEM((chunk, R), jnp.float32),
            pltpu.VMEM_SHARED((rows_pad, R), jnp.float32),
            pltpu.SemaphoreType.DMA,
            pltpu.SemaphoreType.DMA,
            pltpu.SemaphoreType.DMA,
        ],
    )
    def k(z_hbm, src_hbm, dst_hbm, zero_hbm, out_hbm,
          sidx, didx, st0, st1, agg, sem0, sem1, zsem):
        c = lax.axis_index("c")
        s = lax.axis_index("s")
        w = c * _NSUB + s
        zcp = pltpu.make_async_copy(
            zero_hbm.at[pl.ds(s * rpt_zero, rpt_zero)],
            agg.at[pl.ds(s * rpt_zero, rpt_zero)], zsem)
        zcp.start()
        pltpu.sync_copy(src_hbm.at[w], sidx)
        pltpu.sync_copy(dst_hbm.at[w], didx)
        zcp.wait()
        plsc.subcore_barrier()

        def gather(kk, st, sem):
            return pltpu.make_async_copy(z_hbm.at[sidx.at[kk]], st, sem)

        gather(0, st0, sem0).start()

        @pl.loop(0, nk // 2)
        def _(i):
            k0 = 2 * i
            gather(k0 + 1, st1, sem1).start()
            gather(k0, st0, sem0).wait()
            pltpu.sync_copy(st0, agg.at[didx.at[k0]], add=True)

            @pl.when(k0 + 2 < nk)
            def _():
                gather(k0 + 2, st0, sem0).start()

            gather(k0 + 1, st1, sem1).wait()
            pltpu.sync_copy(st1, agg.at[didx.at[k0 + 1]], add=True)

        plsc.subcore_barrier()
        pltpu.sync_copy(agg.at[pl.ds(s * rpt_out, rpt_out)],
                        out_hbm.at[c, pl.ds(s * rpt_out, rpt_out)])

    return k(z, src_r, dst_r, zeros)


def _mix(p, M, bvec, bn):
    n_out, Rin = p.shape[1], p.shape[2]
    Rout = M.shape[1]

    def body(p_ref, m_ref, b_ref, o_ref):
        a = p_ref[0] + p_ref[1]
        o_ref[...] = (
            jnp.dot(a, m_ref[...], preferred_element_type=jnp.float32)
            + b_ref[...])

    return pl.pallas_call(
        body,
        grid=(n_out // bn,),
        in_specs=[
            pl.BlockSpec((2, bn, Rin), lambda i: (0, i, 0)),
            pl.BlockSpec((Rin, Rout), lambda i: (0, 0)),
            pl.BlockSpec((1, Rout), lambda i: (0, 0)),
        ],
        out_specs=pl.BlockSpec((bn, Rout), lambda i: (i, 0)),
        out_shape=jax.ShapeDtypeStruct((n_out, Rout), jnp.float32),
    )(p, M, bvec)


def _final(p4, M4, bv4, W1T, b1, W2T, b2):
    def body(p_ref, m_ref, bv_ref, w1_ref, b1_ref, w2_ref, b2_ref, o_ref):
        z5 = (jnp.dot(p_ref[0] + p_ref[1], m_ref[...],
                      preferred_element_type=jnp.float32) + bv_ref[...])
        flat = z5.T.reshape(_B, 128 * 128)
        h = (jnp.dot(flat, w1_ref[...], preferred_element_type=jnp.float32)
             + b1_ref[...])
        o_ref[...] = (
            jnp.dot(h, w2_ref[...], preferred_element_type=jnp.float32)
            + b2_ref[...])

    return pl.pallas_call(
        body,
        out_shape=jax.ShapeDtypeStruct((_B, 10), jnp.float32),
    )(p4, M4, bv4, W1T, b1, W2T, b2)


def kernel(x, src0, dst0, W0, b0, src1, dst1, W1, b1, src2, dst2, W2, b2,
           src3, dst3, W3, b3, src4, dst4, W4, b4, W_fc1, b_fc1, W_fc2, b_fc2):
    srcs = [src0, src1, src2, src3, src4]
    dsts = [dst0, dst1, dst2, dst3, dst4]
    Ws = [W0, W1, W2, W3, W4]
    bs = [b0, b1, b2, b3, b4]

    eye = jnp.eye(_B, dtype=jnp.float32)
    z = x.T
    partials = None
    for l in range(5):
        n_in, n_out = _NODE[l], _NODE[l + 1]
        R_in = _B * _CH[l]
        src_r, dst_r, nk, chunk = _prep_edges(srcs[l], dsts[l], n_out)
        zeros = jnp.zeros((n_out + _TRASH, R_in), jnp.float32)
        partials = _sc_gather_scatter(n_in, n_out, R_in, nk, chunk,
                                      z, src_r, dst_r, zeros)
        if l < 4:
            M = jnp.kron(eye, Ws[l].T)
            bvec = jnp.tile(bs[l], _B)[None, :]
            bn = min(n_out, 8192)
            z = _mix(partials, M, bvec, bn)

    M4 = jnp.kron(eye, W4.T)
    bv4 = jnp.tile(b4, _B)[None, :]
    out = _final(partials, M4, bv4,
                 W_fc1.T, b_fc1[None, :], W_fc2.T, b_fc2[None, :])
    return out


# speedup vs baseline: 97.7281x; 1.3921x over previous
import functools

import jax
import jax.numpy as jnp
from jax import lax
from jax.experimental import pallas as pl
from jax.experimental.pallas import tpu as pltpu
from jax.experimental.pallas import tpu_sc as plsc

_NODE = [100000, 65536, 16384, 4096, 1024, 128]
_CH = [1, 8, 16, 32, 64, 128]
_B = 8
_NSUB = 16
_NW = 2 * _NSUB
_CHUNK = 128
_TRASH = 128


def _prep_edges(src, dst, n_out):
    E = src.shape[0]
    per = E // _NW
    chunk = _CHUNK if per >= 2 * _CHUNK else max(8, per // 2)
    nk = -(-per // chunk)
    nk += nk % 2
    pad = nk * chunk - per
    s = src.reshape(_NW, per)
    d = dst.reshape(_NW, per)
    if pad:
        s = jnp.pad(s, ((0, 0), (0, pad)))
        trash = n_out + (jnp.arange(pad, dtype=jnp.int32) % _TRASH)
        d = jnp.concatenate(
            [d, jnp.broadcast_to(trash, (_NW, pad))], axis=1)
    return s.reshape(_NW, nk, chunk), d.reshape(_NW, nk, chunk), nk, chunk


def _sc_gather_scatter(n_in, n_out, R, nk, chunk, z, src_r, dst_r, zeros):
    rows_pad = n_out + _TRASH
    rpt_zero = rows_pad // _NSUB
    rpt_out = n_out // _NSUB
    mesh = plsc.VectorSubcoreMesh(core_axis_name="c", subcore_axis_name="s")

    @functools.partial(
        pl.kernel,
        mesh=mesh,
        compiler_params=pltpu.CompilerParams(use_tc_tiling_on_sc=False),
        out_type=jax.ShapeDtypeStruct((2, n_out, R), jnp.float32),
        scratch_types=[
            pltpu.VMEM((nk, chunk), jnp.int32),
            pltpu.VMEM((nk, chunk), jnp.int32),
            pltpu.VMEM((chunk, R), jnp.float32),
            pltpu.VMEM((chunk, R), jnp.float32),
            pltpu.VMEM_SHARED((rows_pad, R), jnp.float32),
            pltpu.SemaphoreType.DMA,
            pltpu.SemaphoreType.DMA,
            pltpu.SemaphoreType.DMA,
        ],
    )
    def k(z_hbm, src_hbm, dst_hbm, zero_hbm, out_hbm,
          sidx, didx, st0, st1, agg, sem0, sem1, zsem):
        c = lax.axis_index("c")
        s = lax.axis_index("s")
        w = c * _NSUB + s
        zcp = pltpu.make_async_copy(
            zero_hbm.at[pl.ds(s * rpt_zero, rpt_zero)],
            agg.at[pl.ds(s * rpt_zero, rpt_zero)], zsem)
        zcp.start()
        pltpu.sync_copy(src_hbm.at[w], sidx)
        pltpu.sync_copy(dst_hbm.at[w], didx)
        zcp.wait()
        plsc.subcore_barrier()

        def gather(kk, st, sem):
            return pltpu.make_async_copy(z_hbm.at[sidx.at[kk]], st, sem)

        gather(0, st0, sem0).start()

        @pl.loop(0, nk // 2)
        def _(i):
            k0 = 2 * i
            gather(k0 + 1, st1, sem1).start()
            gather(k0, st0, sem0).wait()
            pltpu.sync_copy(st0, agg.at[didx.at[k0]], add=True)

            @pl.when(k0 + 2 < nk)
            def _():
                gather(k0 + 2, st0, sem0).start()

            gather(k0 + 1, st1, sem1).wait()
            pltpu.sync_copy(st1, agg.at[didx.at[k0 + 1]], add=True)

        plsc.subcore_barrier()
        pltpu.sync_copy(agg.at[pl.ds(s * rpt_out, rpt_out)],
                        out_hbm.at[c, pl.ds(s * rpt_out, rpt_out)])

    return k(z, src_r, dst_r, zeros)


def _mix(p, W, b, n_out):
    M = jnp.kron(jnp.eye(_B, dtype=jnp.float32), W.T)
    Rin, Rout = M.shape
    g = max(1, 128 // Rin)
    ki = max(1, Rin // 128)
    s = g * Rout // 128
    NR = n_out * Rin // (128 * ki)
    bnr = min(NR, 4096)
    Mfull = jnp.kron(jnp.eye(g, dtype=jnp.float32), M)
    bcat = jnp.tile(jnp.tile(b, _B), g)[None, :]

    def body(p_ref, m_ref, b_ref, o_ref):
        a = p_ref[0] + p_ref[1]
        res = b_ref[...]
        for r in range(ki):
            res = res + jnp.dot(a[:, r, :], m_ref[r * 128:(r + 1) * 128, :],
                                preferred_element_type=jnp.float32)
        for r in range(s):
            o_ref[:, r, :] = res[:, r * 128:(r + 1) * 128]

    z_w = pl.pallas_call(
        body,
        grid=(NR // bnr,),
        in_specs=[
            pl.BlockSpec((2, bnr, ki, 128), lambda i: (0, i, 0, 0)),
            pl.BlockSpec((ki * 128, g * Rout), lambda i: (0, 0)),
            pl.BlockSpec((1, g * Rout), lambda i: (0, 0)),
        ],
        out_specs=pl.BlockSpec((bnr, s, 128), lambda i: (i, 0, 0)),
        out_shape=jax.ShapeDtypeStruct((NR, s, 128), jnp.float32),
    )(p.reshape(2, NR, ki, 128), Mfull, bcat)
    return z_w.reshape(n_out, Rout)


def _final(p4, M4, bv4, W1T, b1, W2T, b2):
    def body(p_ref, m_ref, bv_ref, w1_ref, b1_ref, w2_ref, b2_ref, o_ref):
        a = p_ref[0] + p_ref[1]
        z5 = bv_ref[...]
        for r in range(4):
            z5 = z5 + jnp.dot(a[:, r, :], m_ref[r * 128:(r + 1) * 128, :],
                              preferred_element_type=jnp.float32)
        flat = z5.T.reshape(_B, 128 * 128)
        h = (jnp.dot(flat, w1_ref[...], preferred_element_type=jnp.float32)
             + b1_ref[...])
        o_ref[...] = (
            jnp.dot(h, w2_ref[...], preferred_element_type=jnp.float32)
            + b2_ref[...])

    return pl.pallas_call(
        body,
        out_shape=jax.ShapeDtypeStruct((_B, 10), jnp.float32),
    )(p4, M4, bv4, W1T, b1, W2T, b2)


def kernel(x, src0, dst0, W0, b0, src1, dst1, W1, b1, src2, dst2, W2, b2,
           src3, dst3, W3, b3, src4, dst4, W4, b4, W_fc1, b_fc1, W_fc2, b_fc2):
    srcs = [src0, src1, src2, src3, src4]
    dsts = [dst0, dst1, dst2, dst3, dst4]
    Ws = [W0, W1, W2, W3, W4]
    bs = [b0, b1, b2, b3, b4]

    eye = jnp.eye(_B, dtype=jnp.float32)
    z = x.T
    partials = None
    for l in range(5):
        n_in, n_out = _NODE[l], _NODE[l + 1]
        R_in = _B * _CH[l]
        src_r, dst_r, nk, chunk = _prep_edges(srcs[l], dsts[l], n_out)
        zeros = jnp.zeros((n_out + _TRASH, R_in), jnp.float32)
        partials = _sc_gather_scatter(n_in, n_out, R_in, nk, chunk,
                                      z, src_r, dst_r, zeros)
        if l < 4:
            z = _mix(partials, Ws[l], bs[l], n_out)

    M4 = jnp.kron(eye, W4.T)
    bv4 = jnp.tile(b4, _B)[None, :]
    out = _final(partials.reshape(2, 128, 4, 128), M4, bv4,
                 W_fc1.T, b_fc1[None, :], W_fc2.T, b_fc2[None, :])
    return out


# speedup vs baseline: 105.2749x; 1.0772x over previous
import dataclasses
import functools

import jax
import jax.numpy as jnp
from jax import lax
from jax.experimental import pallas as pl
from jax.experimental.pallas import tpu as pltpu
from jax.experimental.pallas import tpu_sc as plsc

_NODE = [100000, 65536, 16384, 4096, 1024, 128]
_CH = [1, 8, 16, 32, 64, 128]
_B = 8
_NSUB = 16
_NW = 2 * _NSUB
_CHUNK = 128
_TRASH = 128


def _prep_edges(src, dst, n_out, R):
    E = src.shape[0]
    per = E // _NW
    chunk = min(_CHUNK, max(8, 16384 // R))
    if per < 2 * chunk:
        chunk = max(8, per // 2)
    nbuf = 2
    nk = -(-per // chunk)
    if nk >= 4:
        nbuf = 4
        nk = -(-nk // 4) * 4
    else:
        nk += nk % 2
    pad = nk * chunk - per
    s = src.reshape(_NW, per)
    d = dst.reshape(_NW, per)
    if pad:
        s = jnp.pad(s, ((0, 0), (0, pad)))
        trash = n_out + (jnp.arange(pad, dtype=jnp.int32) % _TRASH)
        d = jnp.concatenate(
            [d, jnp.broadcast_to(trash, (_NW, pad))], axis=1)
    return s.reshape(_NW, nk, chunk), d.reshape(_NW, nk, chunk), nk, chunk, nbuf


def _sc_gather_scatter(n_in, n_out, R, nk, chunk, nbuf, z, src_r, dst_r,
                       zeros):
    rows_pad = n_out + _TRASH
    rpt_zero = rows_pad // _NSUB
    rpt_out = n_out // _NSUB
    mesh = plsc.VectorSubcoreMesh(core_axis_name="c", subcore_axis_name="s")

    @functools.partial(
        pl.kernel,
        mesh=mesh,
        compiler_params=pltpu.CompilerParams(use_tc_tiling_on_sc=False),
        out_type=jax.ShapeDtypeStruct((2, n_out, R), jnp.float32),
        scratch_types=[
            pltpu.VMEM((nk, chunk), jnp.int32),
            pltpu.VMEM((nk, chunk), jnp.int32),
        ] + [pltpu.VMEM((chunk, R), jnp.float32)] * nbuf
          + [pltpu.VMEM_SHARED((rows_pad, R), jnp.float32)]
          + [pltpu.SemaphoreType.DMA] * (nbuf + 1),
    )
    def k(z_hbm, src_hbm, dst_hbm, zero_hbm, out_hbm, sidx, didx, *scr):
        sts = scr[:nbuf]
        agg = scr[nbuf]
        sems = scr[nbuf + 1:2 * nbuf + 1]
        zsem = scr[2 * nbuf + 1]
        c = lax.axis_index("c")
        s = lax.axis_index("s")
        w = c * _NSUB + s
        zcp = pltpu.make_async_copy(
            zero_hbm.at[pl.ds(s * rpt_zero, rpt_zero)],
            agg.at[pl.ds(s * rpt_zero, rpt_zero)], zsem)
        zcp.start()
        pltpu.sync_copy(src_hbm.at[w], sidx)
        pltpu.sync_copy(dst_hbm.at[w], didx)
        zcp.wait()
        plsc.subcore_barrier()

        def gather(kk, b):
            return pltpu.make_async_copy(z_hbm.at[sidx.at[kk]],
                                         sts[b], sems[b])

        for j in range(nbuf - 1):
            gather(j, j).start()

        @pl.loop(0, nk // nbuf)
        def _(i):
            for par in range(nbuf):
                kk = i * nbuf + par
                gather(kk, par).wait()
                pltpu.sync_copy(sts[par], agg.at[didx.at[kk]], add=True)
                nxt_b = (par + nbuf - 1) % nbuf

                @pl.when(kk + nbuf - 1 < nk)
                def _():
                    gather(kk + nbuf - 1, nxt_b).start()

        plsc.subcore_barrier()
        pltpu.sync_copy(agg.at[pl.ds(s * rpt_out, rpt_out)],
                        out_hbm.at[c, pl.ds(s * rpt_out, rpt_out)])

    return k(z, src_r, dst_r, zeros)


def _sc_transpose_x(x):
    n0p = x.shape[1]
    per = n0p // _NW
    mesh = plsc.VectorSubcoreMesh(core_axis_name="c", subcore_axis_name="s")
    cp = pltpu.CompilerParams(use_tc_tiling_on_sc=False)
    if "needs_layout_passes" in pltpu.CompilerParams.__dataclass_fields__:
        cp = dataclasses.replace(cp, needs_layout_passes=False)

    @functools.partial(
        pl.kernel,
        mesh=mesh,
        compiler_params=cp,
        out_type=jax.ShapeDtypeStruct((n0p, _B), jnp.float32),
        scratch_types=[
            pltpu.VMEM((_B, per), jnp.float32),
            pltpu.VMEM((per, _B), jnp.float32),
        ],
    )
    def k(x_hbm, out_hbm, xin, xt):
        c = lax.axis_index("c")
        s = lax.axis_index("s")
        w = c * _NSUB + s
        base = w * per
        pltpu.sync_copy(x_hbm.at[:, pl.ds(base, per)], xin)

        @pl.loop(0, per // 16)
        def _(j):
            col0 = j * 16
            rows = col0 + lax.iota(jnp.int32, 16)
            for b in range(_B):
                v = xin[b, pl.ds(col0, 16)]
                plsc.store_scatter(
                    xt, [rows, jnp.full((16,), b, jnp.int32)], v)

        pltpu.sync_copy(xt, out_hbm.at[pl.ds(base, per)])

    return k(x)


def _mix(p, W, b, n_out):
    M = jnp.kron(jnp.eye(_B, dtype=jnp.float32), W.T)
    Rin, Rout = M.shape
    g = max(1, 128 // Rin)
    ki = max(1, Rin // 128)
    s = g * Rout // 128
    NR = n_out * Rin // (128 * ki)
    bnr = min(NR, 4096)
    Mfull = jnp.kron(jnp.eye(g, dtype=jnp.float32), M)
    bcat = jnp.tile(jnp.tile(b, _B), g)[None, :]

    def body(p_ref, m_ref, b_ref, o_ref):
        a = p_ref[0] + p_ref[1]
        res = b_ref[...]
        for r in range(ki):
            res = res + jnp.dot(a[:, r, :], m_ref[r * 128:(r + 1) * 128, :],
                                preferred_element_type=jnp.float32)
        for r in range(s):
            o_ref[:, r, :] = res[:, r * 128:(r + 1) * 128]

    z_w = pl.pallas_call(
        body,
        grid=(NR // bnr,),
        in_specs=[
            pl.BlockSpec((2, bnr, ki, 128), lambda i: (0, i, 0, 0)),
            pl.BlockSpec((ki * 128, g * Rout), lambda i: (0, 0)),
            pl.BlockSpec((1, g * Rout), lambda i: (0, 0)),
        ],
        out_specs=pl.BlockSpec((bnr, s, 128), lambda i: (i, 0, 0)),
        out_shape=jax.ShapeDtypeStruct((NR, s, 128), jnp.float32),
    )(p.reshape(2, NR, ki, 128), Mfull, bcat)
    return z_w.reshape(n_out, Rout)


def _final(p4, M4, bv4, W1T, b1, W2T, b2):
    def body(p_ref, m_ref, bv_ref, w1_ref, b1_ref, w2_ref, b2_ref, o_ref):
        a = p_ref[0] + p_ref[1]
        z5 = bv_ref[...]
        for r in range(4):
            z5 = z5 + jnp.dot(a[:, r, :], m_ref[r * 128:(r + 1) * 128, :],
                              preferred_element_type=jnp.float32)
        flat = z5.T.reshape(_B, 128 * 128)
        h = (jnp.dot(flat, w1_ref[...], preferred_element_type=jnp.float32)
             + b1_ref[...])
        o_ref[...] = (
            jnp.dot(h, w2_ref[...], preferred_element_type=jnp.float32)
            + b2_ref[...])

    return pl.pallas_call(
        body,
        out_shape=jax.ShapeDtypeStruct((_B, 10), jnp.float32),
    )(p4, M4, bv4, W1T, b1, W2T, b2)


def kernel(x, src0, dst0, W0, b0, src1, dst1, W1, b1, src2, dst2, W2, b2,
           src3, dst3, W3, b3, src4, dst4, W4, b4, W_fc1, b_fc1, W_fc2, b_fc2):
    srcs = [src0, src1, src2, src3, src4]
    dsts = [dst0, dst1, dst2, dst3, dst4]
    Ws = [W0, W1, W2, W3, W4]
    bs = [b0, b1, b2, b3, b4]

    eye = jnp.eye(_B, dtype=jnp.float32)
    n0p = 100352
    xp = jnp.pad(x, ((0, 0), (0, n0p - _NODE[0])))
    z = _sc_transpose_x(xp)
    partials = None
    for l in range(5):
        n_in, n_out = _NODE[l], _NODE[l + 1]
        R_in = _B * _CH[l]
        src_r, dst_r, nk, chunk, nbuf = _prep_edges(srcs[l], dsts[l],
                                                    n_out, R_in)
        zeros = jnp.zeros((n_out + _TRASH, R_in), jnp.float32)
        partials = _sc_gather_scatter(n_in, n_out, R_in, nk, chunk, nbuf,
                                      z, src_r, dst_r, zeros)
        if l < 4:
            z = _mix(partials, Ws[l], bs[l], n_out)

    M4 = jnp.kron(eye, W4.T)
    bv4 = jnp.tile(b4, _B)[None, :]
    out = _final(partials.reshape(2, 128, 4, 128), M4, bv4,
                 W_fc1.T, b_fc1[None, :], W_fc2.T, b_fc2[None, :])
    return out


# speedup vs baseline: 106.1925x; 1.0087x over previous
import dataclasses
import functools

import jax
import jax.numpy as jnp
from jax import lax
from jax.experimental import pallas as pl
from jax.experimental.pallas import tpu as pltpu
from jax.experimental.pallas import tpu_sc as plsc

_NODE = [100000, 65536, 16384, 4096, 1024, 128]
_CH = [1, 8, 16, 32, 64, 128]
_B = 8
_NSUB = 16
_NW = 2 * _NSUB
_CHUNK = 128
_TRASH = 128


def _prep_edges(src, dst, n_out, R):
    E = src.shape[0]
    per = E // _NW
    chunk = min(_CHUNK, max(8, 16384 // R))
    if per < 2 * chunk:
        chunk = max(8, per // 2)
    nbuf = 2
    nk = -(-per // chunk)
    if nk >= 4:
        nbuf = 4
        nk = -(-nk // 4) * 4
    else:
        nk += nk % 2
    pad = nk * chunk - per
    s = src.reshape(_NW, per)
    d = dst.reshape(_NW, per)
    if pad:
        s = jnp.pad(s, ((0, 0), (0, pad)))
        trash = n_out + (jnp.arange(pad, dtype=jnp.int32) % _TRASH)
        d = jnp.concatenate(
            [d, jnp.broadcast_to(trash, (_NW, pad))], axis=1)
    return s.reshape(_NW, nk, chunk), d.reshape(_NW, nk, chunk), nk, chunk, nbuf


def _sc_gather_scatter(n_in, n_out, R, nk, chunk, nbuf, z, src_r, dst_r,
                       zeros):
    rows_pad = n_out + _TRASH
    rpt_zero = rows_pad // _NSUB
    rpt_out = n_out // _NSUB
    mesh = plsc.VectorSubcoreMesh(core_axis_name="c", subcore_axis_name="s")

    @functools.partial(
        pl.kernel,
        mesh=mesh,
        compiler_params=pltpu.CompilerParams(use_tc_tiling_on_sc=False),
        out_type=jax.ShapeDtypeStruct((2, n_out, R), jnp.float32),
        scratch_types=[
            pltpu.VMEM((nk, chunk), jnp.int32),
            pltpu.VMEM((nk, chunk), jnp.int32),
        ] + [pltpu.VMEM((chunk, R), jnp.float32)] * nbuf
          + [pltpu.VMEM_SHARED((rows_pad, R), jnp.float32)]
          + [pltpu.SemaphoreType.DMA] * (nbuf + 1),
    )
    def k(z_hbm, src_hbm, dst_hbm, zero_hbm, out_hbm, sidx, didx, *scr):
        sts = scr[:nbuf]
        agg = scr[nbuf]
        sems = scr[nbuf + 1:2 * nbuf + 1]
        zsem = scr[2 * nbuf + 1]
        c = lax.axis_index("c")
        s = lax.axis_index("s")
        w = c * _NSUB + s
        zcp = pltpu.make_async_copy(
            zero_hbm.at[pl.ds(s * rpt_zero, rpt_zero)],
            agg.at[pl.ds(s * rpt_zero, rpt_zero)], zsem)
        zcp.start()
        pltpu.sync_copy(src_hbm.at[w], sidx)
        pltpu.sync_copy(dst_hbm.at[w], didx)
        zcp.wait()
        plsc.subcore_barrier()

        def gather(kk, b):
            return pltpu.make_async_copy(z_hbm.at[sidx.at[kk]],
                                         sts[b], sems[b])

        for j in range(nbuf - 1):
            gather(j, j).start()

        @pl.loop(0, nk // nbuf)
        def _(i):
            for par in range(nbuf):
                kk = i * nbuf + par
                nxt_b = (par + nbuf - 1) % nbuf

                @pl.when(kk + nbuf - 1 < nk)
                def _():
                    gather(kk + nbuf - 1, nxt_b).start()

                gather(kk, par).wait()
                pltpu.sync_copy(sts[par], agg.at[didx.at[kk]], add=True)

        plsc.subcore_barrier()
        pltpu.sync_copy(agg.at[pl.ds(s * rpt_out, rpt_out)],
                        out_hbm.at[c, pl.ds(s * rpt_out, rpt_out)])

    return k(z, src_r, dst_r, zeros)


def _sc_transpose_x(x):
    n0p = x.shape[1]
    per = n0p // _NW
    mesh = plsc.VectorSubcoreMesh(core_axis_name="c", subcore_axis_name="s")
    cp = pltpu.CompilerParams(use_tc_tiling_on_sc=False)
    if "needs_layout_passes" in pltpu.CompilerParams.__dataclass_fields__:
        cp = dataclasses.replace(cp, needs_layout_passes=False)

    @functools.partial(
        pl.kernel,
        mesh=mesh,
        compiler_params=cp,
        out_type=jax.ShapeDtypeStruct((n0p, _B), jnp.float32),
        scratch_types=[
            pltpu.VMEM((_B, per), jnp.float32),
            pltpu.VMEM((per, _B), jnp.float32),
        ],
    )
    def k(x_hbm, out_hbm, xin, xt):
        c = lax.axis_index("c")
        s = lax.axis_index("s")
        w = c * _NSUB + s
        base = w * per
        pltpu.sync_copy(x_hbm.at[:, pl.ds(base, per)], xin)

        @pl.loop(0, per // 16)
        def _(j):
            col0 = j * 16
            rows = col0 + lax.iota(jnp.int32, 16)
            for b in range(_B):
                v = xin[b, pl.ds(col0, 16)]
                plsc.store_scatter(
                    xt, [rows, jnp.full((16,), b, jnp.int32)], v)

        pltpu.sync_copy(xt, out_hbm.at[pl.ds(base, per)])

    return k(x)


def _mix(p, W, b, n_out):
    M = jnp.kron(jnp.eye(_B, dtype=jnp.float32), W.T)
    Rin, Rout = M.shape
    g = max(1, 128 // Rin)
    ki = max(1, Rin // 128)
    s = g * Rout // 128
    NR = n_out * Rin // (128 * ki)
    bnr = min(NR, 4096)
    Mfull = jnp.kron(jnp.eye(g, dtype=jnp.float32), M)
    bcat = jnp.tile(jnp.tile(b, _B), g)[None, :]

    def body(p_ref, m_ref, b_ref, o_ref):
        a = p_ref[0] + p_ref[1]
        res = b_ref[...]
        for r in range(ki):
            res = res + jnp.dot(a[:, r, :], m_ref[r * 128:(r + 1) * 128, :],
                                preferred_element_type=jnp.float32)
        for r in range(s):
            o_ref[:, r, :] = res[:, r * 128:(r + 1) * 128]

    z_w = pl.pallas_call(
        body,
        grid=(NR // bnr,),
        in_specs=[
            pl.BlockSpec((2, bnr, ki, 128), lambda i: (0, i, 0, 0)),
            pl.BlockSpec((ki * 128, g * Rout), lambda i: (0, 0)),
            pl.BlockSpec((1, g * Rout), lambda i: (0, 0)),
        ],
        out_specs=pl.BlockSpec((bnr, s, 128), lambda i: (i, 0, 0)),
        out_shape=jax.ShapeDtypeStruct((NR, s, 128), jnp.float32),
    )(p.reshape(2, NR, ki, 128), Mfull, bcat)
    return z_w.reshape(n_out, Rout)


def _final(p4, M4, bv4, W1T, b1, W2T, b2):
    def body(p_ref, m_ref, bv_ref, w1_ref, b1_ref, w2_ref, b2_ref, o_ref):
        a = p_ref[0] + p_ref[1]
        z5 = bv_ref[...]
        for r in range(4):
            z5 = z5 + jnp.dot(a[:, r, :], m_ref[r * 128:(r + 1) * 128, :],
                              preferred_element_type=jnp.float32)
        flat = z5.T.reshape(_B, 128 * 128)
        h = (jnp.dot(flat, w1_ref[...], preferred_element_type=jnp.float32)
             + b1_ref[...])
        o_ref[...] = (
            jnp.dot(h, w2_ref[...], preferred_element_type=jnp.float32)
            + b2_ref[...])

    return pl.pallas_call(
        body,
        out_shape=jax.ShapeDtypeStruct((_B, 10), jnp.float32),
    )(p4, M4, bv4, W1T, b1, W2T, b2)


def kernel(x, src0, dst0, W0, b0, src1, dst1, W1, b1, src2, dst2, W2, b2,
           src3, dst3, W3, b3, src4, dst4, W4, b4, W_fc1, b_fc1, W_fc2, b_fc2):
    srcs = [src0, src1, src2, src3, src4]
    dsts = [dst0, dst1, dst2, dst3, dst4]
    Ws = [W0, W1, W2, W3, W4]
    bs = [b0, b1, b2, b3, b4]

    eye = jnp.eye(_B, dtype=jnp.float32)
    n0p = 100352
    xp = jnp.pad(x, ((0, 0), (0, n0p - _NODE[0])))
    z = _sc_transpose_x(xp)
    partials = None
    for l in range(5):
        n_in, n_out = _NODE[l], _NODE[l + 1]
        R_in = _B * _CH[l]
        src_r, dst_r, nk, chunk, nbuf = _prep_edges(srcs[l], dsts[l],
                                                    n_out, R_in)
        zeros = jnp.zeros((n_out + _TRASH, R_in), jnp.float32)
        partials = _sc_gather_scatter(n_in, n_out, R_in, nk, chunk, nbuf,
                                      z, src_r, dst_r, zeros)
        if l < 4:
            z = _mix(partials, Ws[l], bs[l], n_out)

    M4 = jnp.kron(eye, W4.T)
    bv4 = jnp.tile(b4, _B)[None, :]
    out = _final(partials.reshape(2, 128, 4, 128), M4, bv4,
                 W_fc1.T, b_fc1[None, :], W_fc2.T, b_fc2[None, :])
    return out


# speedup vs baseline: 120.7472x; 1.1371x over previous
import dataclasses
import functools

import jax
import jax.numpy as jnp
from jax import lax
from jax.experimental import pallas as pl
from jax.experimental.pallas import tpu as pltpu
from jax.experimental.pallas import tpu_sc as plsc

_NODE = [100000, 65536, 16384, 4096, 1024, 128]
_CH = [1, 8, 16, 32, 64, 128]
_B = 8
_NSUB = 16
_NW = 2 * _NSUB
_CHUNK = 128
_TRASH = 128


def _prep_edges(src, dst, n_out, R):
    E = src.shape[0]
    per = E // _NW
    chunk = min(_CHUNK, max(8, 16384 // R))
    if per < 2 * chunk:
        chunk = max(8, per // 2)
    nbuf = 2
    nk = -(-per // chunk)
    if nk >= 4 and R >= 64:
        nbuf = 4
        nk = -(-nk // 4) * 4
    else:
        nk += nk % 2
    pad = nk * chunk - per
    s = src.reshape(_NW, per)
    d = dst.reshape(_NW, per)
    if pad:
        s = jnp.pad(s, ((0, 0), (0, pad)))
        trash = n_out + (jnp.arange(pad, dtype=jnp.int32) % _TRASH)
        d = jnp.concatenate(
            [d, jnp.broadcast_to(trash, (_NW, pad))], axis=1)
    return s.reshape(_NW, nk, chunk), d.reshape(_NW, nk, chunk), nk, chunk, nbuf


def _sc_gather_scatter(n_in, n_out, R, nk, chunk, nbuf, z, src_r, dst_r,
                       zeros):
    rows_pad = n_out + _TRASH
    rpt_zero = rows_pad // _NSUB
    rpt_out = n_out // _NSUB
    mesh = plsc.VectorSubcoreMesh(core_axis_name="c", subcore_axis_name="s")

    @functools.partial(
        pl.kernel,
        mesh=mesh,
        compiler_params=pltpu.CompilerParams(use_tc_tiling_on_sc=False),
        out_type=jax.ShapeDtypeStruct((2, n_out, R), jnp.float32),
        scratch_types=[
            pltpu.VMEM((nk, chunk), jnp.int32),
            pltpu.VMEM((nk, chunk), jnp.int32),
        ] + [pltpu.VMEM((chunk, R), jnp.float32)] * nbuf
          + [pltpu.VMEM_SHARED((rows_pad, R), jnp.float32)]
          + [pltpu.SemaphoreType.DMA] * (nbuf + 1),
    )
    def k(z_hbm, src_hbm, dst_hbm, zero_hbm, out_hbm, sidx, didx, *scr):
        sts = scr[:nbuf]
        agg = scr[nbuf]
        sems = scr[nbuf + 1:2 * nbuf + 1]
        zsem = scr[2 * nbuf + 1]
        c = lax.axis_index("c")
        s = lax.axis_index("s")
        w = c * _NSUB + s
        zcp = pltpu.make_async_copy(
            zero_hbm.at[pl.ds(s * rpt_zero, rpt_zero)],
            agg.at[pl.ds(s * rpt_zero, rpt_zero)], zsem)
        zcp.start()
        pltpu.sync_copy(src_hbm.at[w], sidx)
        pltpu.sync_copy(dst_hbm.at[w], didx)
        zcp.wait()
        plsc.subcore_barrier()

        def gather(kk, b):
            return pltpu.make_async_copy(z_hbm.at[sidx.at[kk]],
                                         sts[b], sems[b])

        for j in range(nbuf - 1):
            gather(j, j).start()

        @pl.loop(0, nk // nbuf)
        def _(i):
            for par in range(nbuf):
                kk = i * nbuf + par
                nxt_b = (par + nbuf - 1) % nbuf

                @pl.when(kk + nbuf - 1 < nk)
                def _():
                    gather(kk + nbuf - 1, nxt_b).start()

                gather(kk, par).wait()
                pltpu.sync_copy(sts[par], agg.at[didx.at[kk]], add=True)

        plsc.subcore_barrier()
        pltpu.sync_copy(agg.at[pl.ds(s * rpt_out, rpt_out)],
                        out_hbm.at[c, pl.ds(s * rpt_out, rpt_out)])

    return k(z, src_r, dst_r, zeros)


def _sc_transpose_x(x):
    n0p = x.shape[1]
    per = n0p // _NW
    mesh = plsc.VectorSubcoreMesh(core_axis_name="c", subcore_axis_name="s")
    cp = pltpu.CompilerParams(use_tc_tiling_on_sc=False)
    if "needs_layout_passes" in pltpu.CompilerParams.__dataclass_fields__:
        cp = dataclasses.replace(cp, needs_layout_passes=False)

    @functools.partial(
        pl.kernel,
        mesh=mesh,
        compiler_params=cp,
        out_type=jax.ShapeDtypeStruct((n0p, _B), jnp.float32),
        scratch_types=[
            pltpu.VMEM((_B, per), jnp.float32),
            pltpu.VMEM((per, _B), jnp.float32),
        ],
    )
    def k(x_hbm, out_hbm, xin, xt):
        c = lax.axis_index("c")
        s = lax.axis_index("s")
        w = c * _NSUB + s
        base = w * per
        pltpu.sync_copy(x_hbm.at[:, pl.ds(base, per)], xin)

        @pl.loop(0, per // 16)
        def _(j):
            col0 = j * 16
            rows = col0 + lax.iota(jnp.int32, 16)
            for b in range(_B):
                v = xin[b, pl.ds(col0, 16)]
                plsc.store_scatter(
                    xt, [rows, jnp.full((16,), b, jnp.int32)], v)

        pltpu.sync_copy(xt, out_hbm.at[pl.ds(base, per)])

    return k(x)


def _mix(p, W, b, n_out):
    M = jnp.kron(jnp.eye(_B, dtype=jnp.float32), W.T)
    Rin, Rout = M.shape
    g = max(1, 128 // Rin)
    ki = max(1, Rin // 128)
    s = g * Rout // 128
    NR = n_out * Rin // (128 * ki)
    bnr = min(NR, 4096)
    Mfull = jnp.kron(jnp.eye(g, dtype=jnp.float32), M)
    bcat = jnp.tile(jnp.tile(b, _B), g)[None, :]

    def body(p_ref, m_ref, b_ref, o_ref):
        a = p_ref[0] + p_ref[1]
        res = b_ref[...]
        for r in range(ki):
            res = res + jnp.dot(a[:, r, :], m_ref[r * 128:(r + 1) * 128, :],
                                preferred_element_type=jnp.float32)
        for r in range(s):
            o_ref[:, r, :] = res[:, r * 128:(r + 1) * 128]

    z_w = pl.pallas_call(
        body,
        grid=(NR // bnr,),
        in_specs=[
            pl.BlockSpec((2, bnr, ki, 128), lambda i: (0, i, 0, 0)),
            pl.BlockSpec((ki * 128, g * Rout), lambda i: (0, 0)),
            pl.BlockSpec((1, g * Rout), lambda i: (0, 0)),
        ],
        out_specs=pl.BlockSpec((bnr, s, 128), lambda i: (i, 0, 0)),
        out_shape=jax.ShapeDtypeStruct((NR, s, 128), jnp.float32),
    )(p.reshape(2, NR, ki, 128), Mfull, bcat)
    return z_w.reshape(n_out, Rout)


def _final(p4, M4, bv4, W1T, b1, W2T, b2):
    def body(p_ref, m_ref, bv_ref, w1_ref, b1_ref, w2_ref, b2_ref, o_ref):
        a = p_ref[0] + p_ref[1]
        z5 = bv_ref[...]
        for r in range(4):
            z5 = z5 + jnp.dot(a[:, r, :], m_ref[r * 128:(r + 1) * 128, :],
                              preferred_element_type=jnp.float32)
        flat = z5.T.reshape(_B, 128 * 128)
        h = (jnp.dot(flat, w1_ref[...], preferred_element_type=jnp.float32)
             + b1_ref[...])
        o_ref[...] = (
            jnp.dot(h, w2_ref[...], preferred_element_type=jnp.float32)
            + b2_ref[...])

    return pl.pallas_call(
        body,
        out_shape=jax.ShapeDtypeStruct((_B, 10), jnp.float32),
    )(p4, M4, bv4, W1T, b1, W2T, b2)


def kernel(x, src0, dst0, W0, b0, src1, dst1, W1, b1, src2, dst2, W2, b2,
           src3, dst3, W3, b3, src4, dst4, W4, b4, W_fc1, b_fc1, W_fc2, b_fc2):
    srcs = [src0, src1, src2, src3, src4]
    dsts = [dst0, dst1, dst2, dst3, dst4]
    Ws = [W0, W1, W2, W3, W4]
    bs = [b0, b1, b2, b3, b4]

    eye = jnp.eye(_B, dtype=jnp.float32)
    n0p = 100352
    xp = jnp.pad(x, ((0, 0), (0, n0p - _NODE[0])))
    z = _sc_transpose_x(xp)
    partials = None
    for l in range(5):
        n_in, n_out = _NODE[l], _NODE[l + 1]
        R_in = _B * _CH[l]
        src_r, dst_r, nk, chunk, nbuf = _prep_edges(srcs[l], dsts[l],
                                                    n_out, R_in)
        zeros = jnp.zeros((n_out + _TRASH, R_in), jnp.float32)
        partials = _sc_gather_scatter(n_in, n_out, R_in, nk, chunk, nbuf,
                                      z, src_r, dst_r, zeros)
        if l < 4:
            z = _mix(partials, Ws[l], bs[l], n_out)

    M4 = jnp.kron(eye, W4.T)
    bv4 = jnp.tile(b4, _B)[None, :]
    out = _final(partials.reshape(2, 128, 4, 128), M4, bv4,
                 W_fc1.T, b_fc1[None, :], W_fc2.T, b_fc2[None, :])
    return out


# speedup vs baseline: 139.6287x; 1.1564x over previous
import dataclasses
import functools

import jax
import jax.numpy as jnp
from jax import lax
from jax.experimental import pallas as pl
from jax.experimental.pallas import tpu as pltpu
from jax.experimental.pallas import tpu_sc as plsc

_NODE = [100000, 65536, 16384, 4096, 1024, 128]
_CH = [1, 8, 16, 32, 64, 128]
_B = 8
_NSUB = 16
_NW = 2 * _NSUB
_CHUNK = 128
_TRASH = 128


def _prep_edges(src, dst, n_out, R):
    E = src.shape[0]
    per = E // _NW
    chunk = min(_CHUNK, max(8, 16384 // R))
    if per < 2 * chunk:
        chunk = max(8, per // 2)
    nbuf = 2
    nk = -(-per // chunk)
    if nk >= 4 and R >= 64:
        nbuf = 4
        nk = -(-nk // 4) * 4
    else:
        nk += nk % 2
    pad = nk * chunk - per
    s = src.reshape(_NW, per)
    d = dst.reshape(_NW, per)
    if pad:
        s = jnp.pad(s, ((0, 0), (0, pad)))
        trash = n_out + (jnp.arange(pad, dtype=jnp.int32) % _TRASH)
        d = jnp.concatenate(
            [d, jnp.broadcast_to(trash, (_NW, pad))], axis=1)
    return s.reshape(_NW, nk, chunk), d.reshape(_NW, nk, chunk), nk, chunk, nbuf


def _sc_gather_scatter(n_in, n_out, R, nk, chunk, nbuf, z, src_r, dst_r,
                       zeros):
    rows_pad = n_out + _TRASH
    rpt_zero = rows_pad // _NSUB
    rpt_out = n_out // _NSUB
    spmem_src = R == 8
    mesh = plsc.VectorSubcoreMesh(core_axis_name="c", subcore_axis_name="s")

    @functools.partial(
        pl.kernel,
        mesh=mesh,
        compiler_params=pltpu.CompilerParams(use_tc_tiling_on_sc=False),
        out_type=jax.ShapeDtypeStruct((2, n_out, R), jnp.float32),
        scratch_types=[
            pltpu.VMEM((nk, chunk), jnp.int32),
            pltpu.VMEM((nk, chunk), jnp.int32),
        ] + [pltpu.VMEM((chunk, R), jnp.float32)] * nbuf
          + [pltpu.VMEM_SHARED((rows_pad, R), jnp.float32)]
          + ([pltpu.VMEM_SHARED((n_in, R), jnp.float32)] if spmem_src else [])
          + [pltpu.SemaphoreType.DMA] * (nbuf + 1),
    )
    def k(z_hbm, src_hbm, dst_hbm, zero_hbm, out_hbm, sidx, didx, *scr):
        sts = scr[:nbuf]
        agg = scr[nbuf]
        off = nbuf + 1
        if spmem_src:
            ztab = scr[off]
            off += 1
        sems = scr[off:off + nbuf]
        zsem = scr[off + nbuf]
        c = lax.axis_index("c")
        s = lax.axis_index("s")
        w = c * _NSUB + s
        zcp = pltpu.make_async_copy(
            zero_hbm.at[pl.ds(s * rpt_zero, rpt_zero)],
            agg.at[pl.ds(s * rpt_zero, rpt_zero)], zsem)
        zcp.start()
        pltpu.sync_copy(src_hbm.at[w], sidx)
        pltpu.sync_copy(dst_hbm.at[w], didx)
        if spmem_src:
            zpt = n_in // _NSUB
            pltpu.sync_copy(z_hbm.at[pl.ds(s * zpt, zpt)],
                            ztab.at[pl.ds(s * zpt, zpt)])
        zcp.wait()
        plsc.subcore_barrier()
        zsrc = ztab if spmem_src else z_hbm

        def gather(kk, b):
            return pltpu.make_async_copy(zsrc.at[sidx.at[kk]],
                                         sts[b], sems[b])

        for j in range(nbuf - 1):
            gather(j, j).start()

        @pl.loop(0, nk // nbuf)
        def _(i):
            for par in range(nbuf):
                kk = i * nbuf + par
                nxt_b = (par + nbuf - 1) % nbuf

                @pl.when(kk + nbuf - 1 < nk)
                def _():
                    gather(kk + nbuf - 1, nxt_b).start()

                gather(kk, par).wait()
                pltpu.sync_copy(sts[par], agg.at[didx.at[kk]], add=True)

        plsc.subcore_barrier()
        pltpu.sync_copy(agg.at[pl.ds(s * rpt_out, rpt_out)],
                        out_hbm.at[c, pl.ds(s * rpt_out, rpt_out)])

    return k(z, src_r, dst_r, zeros)


def _sc_transpose_x(x):
    n0p = x.shape[1]
    per = n0p // _NW
    mesh = plsc.VectorSubcoreMesh(core_axis_name="c", subcore_axis_name="s")
    cp = pltpu.CompilerParams(use_tc_tiling_on_sc=False)
    if "needs_layout_passes" in pltpu.CompilerParams.__dataclass_fields__:
        cp = dataclasses.replace(cp, needs_layout_passes=False)

    @functools.partial(
        pl.kernel,
        mesh=mesh,
        compiler_params=cp,
        out_type=jax.ShapeDtypeStruct((n0p, _B), jnp.float32),
        scratch_types=[
            pltpu.VMEM((_B, per), jnp.float32),
            pltpu.VMEM((per, _B), jnp.float32),
        ],
    )
    def k(x_hbm, out_hbm, xin, xt):
        c = lax.axis_index("c")
        s = lax.axis_index("s")
        w = c * _NSUB + s
        base = w * per
        pltpu.sync_copy(x_hbm.at[:, pl.ds(base, per)], xin)

        @pl.loop(0, per // 16)
        def _(j):
            col0 = j * 16
            rows = col0 + lax.iota(jnp.int32, 16)
            for b in range(_B):
                v = xin[b, pl.ds(col0, 16)]
                plsc.store_scatter(
                    xt, [rows, jnp.full((16,), b, jnp.int32)], v)

        pltpu.sync_copy(xt, out_hbm.at[pl.ds(base, per)])

    return k(x)


def _mix(p, W, b, n_out):
    M = jnp.kron(jnp.eye(_B, dtype=jnp.float32), W.T)
    Rin, Rout = M.shape
    g = max(1, 128 // Rin)
    ki = max(1, Rin // 128)
    s = g * Rout // 128
    NR = n_out * Rin // (128 * ki)
    bnr = min(NR, 4096)
    Mfull = jnp.kron(jnp.eye(g, dtype=jnp.float32), M)
    bcat = jnp.tile(jnp.tile(b, _B), g)[None, :]
    def body(p_ref, m_ref, b_ref, o_ref):
        a = p_ref[0] + p_ref[1]
        res = b_ref[...]
        for r in range(ki):
            res = res + jnp.dot(a[:, r, :], m_ref[r * 128:(r + 1) * 128, :],
                                preferred_element_type=jnp.float32)
        for r in range(s):
            o_ref[:, r, :] = res[:, r * 128:(r + 1) * 128]

    z_w = pl.pallas_call(
        body,
        grid=(NR // bnr,),
        in_specs=[
            pl.BlockSpec((2, bnr, ki, 128), lambda i: (0, i, 0, 0)),
            pl.BlockSpec((ki * 128, g * Rout), lambda i: (0, 0)),
            pl.BlockSpec((1, g * Rout), lambda i: (0, 0)),
        ],
        out_specs=pl.BlockSpec((bnr, s, 128), lambda i: (i, 0, 0)),
        out_shape=jax.ShapeDtypeStruct((NR, s, 128), jnp.float32),
    )(p.reshape(2, NR, ki, 128), Mfull, bcat)
    return z_w.reshape(n_out, Rout)


def _final(p4, M4, bv4, W1T, b1, W2T, b2):
    def body(p_ref, m_ref, bv_ref, w1_ref, b1_ref, w2_ref, b2_ref, o_ref):
        a = p_ref[0] + p_ref[1]
        z5 = bv_ref[...]
        for r in range(4):
            z5 = z5 + jnp.dot(a[:, r, :], m_ref[r * 128:(r + 1) * 128, :],
                              preferred_element_type=jnp.float32)
        flat = z5.T.reshape(_B, 128 * 128)
        h = (jnp.dot(flat, w1_ref[...], preferred_element_type=jnp.float32)
             + b1_ref[...])
        o_ref[...] = (
            jnp.dot(h, w2_ref[...], preferred_element_type=jnp.float32)
            + b2_ref[...])

    return pl.pallas_call(
        body,
        out_shape=jax.ShapeDtypeStruct((_B, 10), jnp.float32),
    )(p4, M4, bv4, W1T, b1, W2T, b2)


def kernel(x, src0, dst0, W0, b0, src1, dst1, W1, b1, src2, dst2, W2, b2,
           src3, dst3, W3, b3, src4, dst4, W4, b4, W_fc1, b_fc1, W_fc2, b_fc2):
    srcs = [src0, src1, src2, src3, src4]
    dsts = [dst0, dst1, dst2, dst3, dst4]
    Ws = [W0, W1, W2, W3, W4]
    bs = [b0, b1, b2, b3, b4]

    eye = jnp.eye(_B, dtype=jnp.float32)
    n0p = 100352
    xp = jnp.pad(x, ((0, 0), (0, n0p - _NODE[0])))
    z = _sc_transpose_x(xp)
    partials = None
    for l in range(5):
        n_in, n_out = _NODE[l], _NODE[l + 1]
        R_in = _B * _CH[l]
        src_r, dst_r, nk, chunk, nbuf = _prep_edges(srcs[l], dsts[l],
                                                    n_out, R_in)
        zeros = jnp.zeros((n_out + _TRASH, R_in), jnp.float32)
        partials = _sc_gather_scatter(n_in, n_out, R_in, nk, chunk, nbuf,
                                      z, src_r, dst_r, zeros)
        if l < 4:
            z = _mix(partials, Ws[l], bs[l], n_out)

    M4 = jnp.kron(eye, W4.T)
    bv4 = jnp.tile(b4, _B)[None, :]
    out = _final(partials.reshape(2, 128, 4, 128), M4, bv4,
                 W_fc1.T, b_fc1[None, :], W_fc2.T, b_fc2[None, :])
    return out


# speedup vs baseline: 139.9292x; 1.0022x over previous
import dataclasses
import functools

import jax
import jax.numpy as jnp
from jax import lax
from jax.experimental import pallas as pl
from jax.experimental.pallas import tpu as pltpu
from jax.experimental.pallas import tpu_sc as plsc

_NODE = [100000, 65536, 16384, 4096, 1024, 128]
_CH = [1, 8, 16, 32, 64, 128]
_B = 8
_NSUB = 16
_NW = 2 * _NSUB
_CHUNK = 128
_TRASH = 128


def _prep_edges(src, dst, n_out, R):
    E = src.shape[0]
    per = E // _NW
    chunk = min(_CHUNK, max(8, 16384 // R))
    if per < 2 * chunk:
        chunk = max(8, per // 2)
    nbuf = 2
    nk = -(-per // chunk)
    if nk >= 4 and R >= 64:
        nbuf = 4
        nk = -(-nk // 4) * 4
    else:
        nk += nk % 2
    pad = nk * chunk - per
    s = src.reshape(_NW, per)
    d = dst.reshape(_NW, per)
    if pad:
        s = jnp.pad(s, ((0, 0), (0, pad)))
        trash = n_out + (jnp.arange(pad, dtype=jnp.int32) % _TRASH)
        d = jnp.concatenate(
            [d, jnp.broadcast_to(trash, (_NW, pad))], axis=1)
    return s.reshape(_NW, nk, chunk), d.reshape(_NW, nk, chunk), nk, chunk, nbuf


def _sc_gather_scatter(n_in, n_out, R, nk, chunk, nbuf, z, src_r, dst_r,
                       zeros):
    rows_pad = n_out + _TRASH
    rpt_zero = rows_pad // _NSUB
    rpt_out = n_out // _NSUB
    spmem_src = R == 8
    mesh = plsc.VectorSubcoreMesh(core_axis_name="c", subcore_axis_name="s")

    @functools.partial(
        pl.kernel,
        mesh=mesh,
        compiler_params=pltpu.CompilerParams(use_tc_tiling_on_sc=False),
        out_type=jax.ShapeDtypeStruct((2, n_out, R), jnp.float32),
        scratch_types=[
            pltpu.VMEM((nk, chunk), jnp.int32),
            pltpu.VMEM((nk, chunk), jnp.int32),
        ] + [pltpu.VMEM((chunk, R), jnp.float32)] * nbuf
          + [pltpu.VMEM_SHARED((rows_pad, R), jnp.float32)]
          + ([pltpu.VMEM_SHARED((n_in, R), jnp.float32)] if spmem_src else [])
          + [pltpu.SemaphoreType.DMA] * (2 * nbuf + 1),
    )
    def k(z_hbm, src_hbm, dst_hbm, zero_hbm, out_hbm, sidx, didx, *scr):
        sts = scr[:nbuf]
        agg = scr[nbuf]
        off = nbuf + 1
        if spmem_src:
            ztab = scr[off]
            off += 1
        sems = scr[off:off + nbuf]
        ssems = scr[off + nbuf:off + 2 * nbuf]
        zsem = scr[off + 2 * nbuf]
        c = lax.axis_index("c")
        s = lax.axis_index("s")
        w = c * _NSUB + s
        zcp = pltpu.make_async_copy(
            zero_hbm.at[pl.ds(s * rpt_zero, rpt_zero)],
            agg.at[pl.ds(s * rpt_zero, rpt_zero)], zsem)
        zcp.start()
        pltpu.sync_copy(src_hbm.at[w], sidx)
        pltpu.sync_copy(dst_hbm.at[w], didx)
        if spmem_src:
            zpt = n_in // _NSUB
            pltpu.sync_copy(z_hbm.at[pl.ds(s * zpt, zpt)],
                            ztab.at[pl.ds(s * zpt, zpt)])
        zcp.wait()
        plsc.subcore_barrier()
        zsrc = ztab if spmem_src else z_hbm

        def gather(kk, b):
            return pltpu.make_async_copy(zsrc.at[sidx.at[kk]],
                                         sts[b], sems[b])

        def scatter_start(kk, b):
            pltpu.async_copy(sts[b], agg.at[didx.at[kk]], ssems[b], add=True)

        def scatter_wait(kk, b):
            pltpu.make_async_copy(sts[b], agg.at[didx.at[kk]],
                                  ssems[b]).wait()

        for j in range(nbuf - 1):
            gather(j, j).start()

        @pl.loop(0, nk // nbuf)
        def _(i):
            for par in range(nbuf):
                kk = i * nbuf + par
                nxt_b = (par + nbuf - 1) % nbuf

                @pl.when((kk + nbuf - 1 < nk) & (kk > 0))
                def _():
                    scatter_wait(kk - 1, nxt_b)
                    gather(kk + nbuf - 1, nxt_b).start()

                @pl.when((kk + nbuf - 1 < nk) & (kk == 0))
                def _():
                    gather(kk + nbuf - 1, nxt_b).start()

                gather(kk, par).wait()
                scatter_start(kk, par)

        for par in range(nbuf):
            scatter_wait(nk - nbuf + par, par)
        plsc.subcore_barrier()
        pltpu.sync_copy(agg.at[pl.ds(s * rpt_out, rpt_out)],
                        out_hbm.at[c, pl.ds(s * rpt_out, rpt_out)])

    return k(z, src_r, dst_r, zeros)


def _sc_transpose_x(x):
    n0p = x.shape[1]
    per = n0p // _NW
    mesh = plsc.VectorSubcoreMesh(core_axis_name="c", subcore_axis_name="s")
    cp = pltpu.CompilerParams(use_tc_tiling_on_sc=False)
    if "needs_layout_passes" in pltpu.CompilerParams.__dataclass_fields__:
        cp = dataclasses.replace(cp, needs_layout_passes=False)

    @functools.partial(
        pl.kernel,
        mesh=mesh,
        compiler_params=cp,
        out_type=jax.ShapeDtypeStruct((n0p, _B), jnp.float32),
        scratch_types=[
            pltpu.VMEM((_B, per), jnp.float32),
            pltpu.VMEM((per, _B), jnp.float32),
        ],
    )
    def k(x_hbm, out_hbm, xin, xt):
        c = lax.axis_index("c")
        s = lax.axis_index("s")
        w = c * _NSUB + s
        base = w * per
        pltpu.sync_copy(x_hbm.at[:, pl.ds(base, per)], xin)

        @pl.loop(0, per // 16)
        def _(j):
            col0 = j * 16
            rows = col0 + lax.iota(jnp.int32, 16)
            for b in range(_B):
                v = xin[b, pl.ds(col0, 16)]
                plsc.store_scatter(
                    xt, [rows, jnp.full((16,), b, jnp.int32)], v)

        pltpu.sync_copy(xt, out_hbm.at[pl.ds(base, per)])

    return k(x)


def _mix(p, W, b, n_out):
    M = jnp.kron(jnp.eye(_B, dtype=jnp.float32), W.T)
    Rin, Rout = M.shape
    g = max(1, 128 // Rin)
    ki = max(1, Rin // 128)
    s = g * Rout // 128
    NR = n_out * Rin // (128 * ki)
    bnr = min(NR, 4096)
    Mfull = jnp.kron(jnp.eye(g, dtype=jnp.float32), M)
    bcat = jnp.tile(jnp.tile(b, _B), g)[None, :]
    def body(p_ref, m_ref, b_ref, o_ref):
        a = p_ref[0] + p_ref[1]
        res = b_ref[...]
        for r in range(ki):
            res = res + jnp.dot(a[:, r, :], m_ref[r * 128:(r + 1) * 128, :],
                                preferred_element_type=jnp.float32)
        for r in range(s):
            o_ref[:, r, :] = res[:, r * 128:(r + 1) * 128]

    z_w = pl.pallas_call(
        body,
        grid=(NR // bnr,),
        in_specs=[
            pl.BlockSpec((2, bnr, ki, 128), lambda i: (0, i, 0, 0)),
            pl.BlockSpec((ki * 128, g * Rout), lambda i: (0, 0)),
            pl.BlockSpec((1, g * Rout), lambda i: (0, 0)),
        ],
        out_specs=pl.BlockSpec((bnr, s, 128), lambda i: (i, 0, 0)),
        out_shape=jax.ShapeDtypeStruct((NR, s, 128), jnp.float32),
    )(p.reshape(2, NR, ki, 128), Mfull, bcat)
    return z_w.reshape(n_out, Rout)


def _final(p4, M4, bv4, W1T, b1, W2T, b2):
    def body(p_ref, m_ref, bv_ref, w1_ref, b1_ref, w2_ref, b2_ref, o_ref):
        a = p_ref[0] + p_ref[1]
        z5 = bv_ref[...]
        for r in range(4):
            z5 = z5 + jnp.dot(a[:, r, :], m_ref[r * 128:(r + 1) * 128, :],
                              preferred_element_type=jnp.float32)
        flat = z5.T.reshape(_B, 128 * 128)
        h = (jnp.dot(flat, w1_ref[...], preferred_element_type=jnp.float32)
             + b1_ref[...])
        o_ref[...] = (
            jnp.dot(h, w2_ref[...], preferred_element_type=jnp.float32)
            + b2_ref[...])

    return pl.pallas_call(
        body,
        out_shape=jax.ShapeDtypeStruct((_B, 10), jnp.float32),
    )(p4, M4, bv4, W1T, b1, W2T, b2)


def kernel(x, src0, dst0, W0, b0, src1, dst1, W1, b1, src2, dst2, W2, b2,
           src3, dst3, W3, b3, src4, dst4, W4, b4, W_fc1, b_fc1, W_fc2, b_fc2):
    srcs = [src0, src1, src2, src3, src4]
    dsts = [dst0, dst1, dst2, dst3, dst4]
    Ws = [W0, W1, W2, W3, W4]
    bs = [b0, b1, b2, b3, b4]

    eye = jnp.eye(_B, dtype=jnp.float32)
    n0p = 100352
    xp = jnp.pad(x, ((0, 0), (0, n0p - _NODE[0])))
    z = _sc_transpose_x(xp)
    partials = None
    for l in range(5):
        n_in, n_out = _NODE[l], _NODE[l + 1]
        R_in = _B * _CH[l]
        src_r, dst_r, nk, chunk, nbuf = _prep_edges(srcs[l], dsts[l],
                                                    n_out, R_in)
        zeros = jnp.zeros((n_out + _TRASH, R_in), jnp.float32)
        partials = _sc_gather_scatter(n_in, n_out, R_in, nk, chunk, nbuf,
                                      z, src_r, dst_r, zeros)
        if l < 4:
            z = _mix(partials, Ws[l], bs[l], n_out)

    M4 = jnp.kron(eye, W4.T)
    bv4 = jnp.tile(b4, _B)[None, :]
    out = _final(partials.reshape(2, 128, 4, 128), M4, bv4,
                 W_fc1.T, b_fc1[None, :], W_fc2.T, b_fc2[None, :])
    return out
